# baseline (speedup 1.0000x reference)
import functools
import math

import jax
import jax.numpy as jnp
from jax import lax
from jax.experimental import pallas as pl
from jax.experimental.pallas import tpu as pltpu
from jax.experimental.pallas import tpu_sc as plsc

D_MODEL = 64
SCALE = math.sqrt(D_MODEL)
LANES = 16

_NC = 2
_NS = 16
_NW = _NC * _NS

_CHUNK_ROWS = 8
_IDX_MINOR = 128


def _emb_kernel(n_rows_total, lut_hbm, idx_hbm, out_hbm, idx_v, rows_v, sem):
    wid = lax.axis_index("s") * _NC + lax.axis_index("c")
    rows_per_w = n_rows_total // _NW
    n_chunks = rows_per_w // _CHUNK_ROWS
    chunk_elems = _CHUNK_ROWS * _IDX_MINOR

    def chunk_body(ch, _):
        row0 = wid * rows_per_w + ch * _CHUNK_ROWS
        pltpu.sync_copy(idx_hbm.at[pl.ds(row0, _CHUNK_ROWS)], idx_v)
        copies = [
            pltpu.async_copy(
                lut_hbm.at[idx_v.at[j]],
                rows_v.at[pl.ds(j * _IDX_MINOR, _IDX_MINOR)],
                sem,
            )
            for j in range(_CHUNK_ROWS)
        ]
        for c in copies:
            c.wait()

        def scale_body(r, _):
            for u in range(8):
                rr = r * 8 + u
                for c in range(D_MODEL // LANES):
                    sl = pl.ds(c * LANES, LANES)
                    rows_v[rr, sl] = rows_v[rr, sl] * SCALE
            return _

        lax.fori_loop(0, chunk_elems // 8, scale_body, 0, unroll=False)

        pltpu.sync_copy(rows_v, out_hbm.at[pl.ds(row0 * _IDX_MINOR, chunk_elems)])
        return _

    lax.fori_loop(0, n_chunks, chunk_body, 0, unroll=False)


def kernel(x, lut):
    b, t = x.shape
    n = b * t
    assert n % (_NW * _CHUNK_ROWS * _IDX_MINOR) == 0
    n_rows_total = n // _IDX_MINOR
    idx2d = x.reshape(n_rows_total, _IDX_MINOR).astype(jnp.int32)

    mesh = plsc.VectorSubcoreMesh(core_axis_name="c", subcore_axis_name="s")
    run = pl.kernel(
        functools.partial(_emb_kernel, n_rows_total),
        out_type=jax.ShapeDtypeStruct((n, D_MODEL), jnp.float32),
        mesh=mesh,
        scratch_types=[
            pltpu.VMEM((_CHUNK_ROWS, _IDX_MINOR), jnp.int32),
            pltpu.VMEM((_CHUNK_ROWS * _IDX_MINOR, D_MODEL), jnp.float32),
            pltpu.SemaphoreType.DMA,
        ],
        compiler_params=pltpu.CompilerParams(use_tc_tiling_on_sc=False),
    )
    out = run(lut, idx2d)
    return out.reshape(b, t, D_MODEL)


# speedup vs baseline: 1.0488x; 1.0488x over previous
import functools
import math

import jax
import jax.numpy as jnp
from jax import lax
from jax.experimental import pallas as pl
from jax.experimental.pallas import tpu as pltpu
from jax.experimental.pallas import tpu_sc as plsc

D_MODEL = 64
SCALE = math.sqrt(D_MODEL)
LANES = 16

_NC = 2
_NS = 16
_NW = _NC * _NS

_CHUNK_ROWS = 4
_IDX_MINOR = 128
_CHUNK = _CHUNK_ROWS * _IDX_MINOR


def _emb_kernel(n_rows_total, lut_hbm, idx_hbm, out_hbm,
                idx0, idx1, rows0, rows1, sem_g, sem_s):
    wid = lax.axis_index("s") * _NC + lax.axis_index("c")
    rows_per_w = n_rows_total // _NW
    n_chunks = rows_per_w // _CHUNK_ROWS
    base_row = wid * rows_per_w
    idx_bufs = (idx0, idx1)
    row_bufs = (rows0, rows1)

    def stage_and_fire(ch, b):
        row0 = base_row + ch * _CHUNK_ROWS
        pltpu.sync_copy(idx_hbm.at[pl.ds(row0, _CHUNK_ROWS)], idx_bufs[b])
        for j in range(_CHUNK_ROWS):
            pltpu.async_copy(
                lut_hbm.at[idx_bufs[b].at[j]],
                row_bufs[b].at[pl.ds(j * _IDX_MINOR, _IDX_MINOR)],
                sem_g,
            )

    def wait_gather(b):
        for j in range(_CHUNK_ROWS):
            pltpu.make_async_copy(
                lut_hbm.at[idx_bufs[b].at[j]],
                row_bufs[b].at[pl.ds(j * _IDX_MINOR, _IDX_MINOR)],
                sem_g,
            ).wait()

    def scale(b):
        rv = row_bufs[b]

        def body(r, carry):
            for u in range(8):
                rr = r * 8 + u
                for c in range(D_MODEL // LANES):
                    sl = pl.ds(c * LANES, LANES)
                    rv[rr, sl] = rv[rr, sl] * SCALE
            return carry

        lax.fori_loop(0, _CHUNK // 8, body, 0)

    def fire_store(ch, b):
        row0 = base_row + ch * _CHUNK_ROWS
        pltpu.async_copy(
            row_bufs[b], out_hbm.at[pl.ds(row0 * _IDX_MINOR, _CHUNK)], sem_s)

    def wait_store(b):
        pltpu.make_async_copy(
            row_bufs[b], out_hbm.at[pl.ds(0, _CHUNK)], sem_s).wait()

    stage_and_fire(0, 0)

    def outer(g, carry):
        for b in (0, 1):
            ch = 2 * g + b
            nxt = ch + 1

            @pl.when(nxt < n_chunks)
            def _():
                @pl.when(ch >= 1)
                def _():
                    wait_store(1 - b)

                stage_and_fire(nxt, 1 - b)

            wait_gather(b)
            scale(b)
            fire_store(ch, b)
        return carry

    lax.fori_loop(0, n_chunks // 2, outer, 0)
    wait_store(0)
    wait_store(1)


def kernel(x, lut):
    b, t = x.shape
    n = b * t
    assert n % (_NW * _CHUNK) == 0
    n_rows_total = n // _IDX_MINOR
    idx2d = x.reshape(n_rows_total, _IDX_MINOR).astype(jnp.int32)

    mesh = plsc.VectorSubcoreMesh(core_axis_name="c", subcore_axis_name="s")
    run = pl.kernel(
        functools.partial(_emb_kernel, n_rows_total),
        out_type=jax.ShapeDtypeStruct((n, D_MODEL), jnp.float32),
        mesh=mesh,
        scratch_types=[
            pltpu.VMEM((_CHUNK_ROWS, _IDX_MINOR), jnp.int32),
            pltpu.VMEM((_CHUNK_ROWS, _IDX_MINOR), jnp.int32),
            pltpu.VMEM((_CHUNK, D_MODEL), jnp.float32),
            pltpu.VMEM((_CHUNK, D_MODEL), jnp.float32),
            pltpu.SemaphoreType.DMA,
            pltpu.SemaphoreType.DMA,
        ],
        compiler_params=pltpu.CompilerParams(use_tc_tiling_on_sc=False),
    )
    out = run(lut, idx2d)
    return out.reshape(b, t, D_MODEL)
